# Initial kernel scaffold; baseline (speedup 1.0000x reference)
import jax
import jax.numpy as jnp
from jax.experimental import pallas as pl


def kernel(x, W, b):
    raise NotImplementedError("write your pallas kernel here")


# speedup vs baseline: 10.8953x; 10.8953x over previous
import jax
import jax.numpy as jnp
import numpy as np
from jax.experimental import pallas as pl

_TOPK = 32
_N = 2048
_NF = _N // 2 + 1
_FP = 1040
_ST = 512
_CT = 256

_PREC_HI = jax.lax.Precision.HIGHEST
_PREC_MED = jax.lax.Precision.HIGHEST


def _basis():
    f = jnp.arange(_FP, dtype=jnp.int32)[:, None]
    s = jnp.arange(_N, dtype=jnp.int32)[None, :]
    m = (f * s) % _N
    ang = m.astype(jnp.float32) * np.float32(2.0 * np.pi / _N)
    valid = (f < _NF).astype(jnp.float32)
    return jnp.cos(ang) * valid, jnp.sin(ang) * valid


def _proj_kernel(x_ref, wt_ref, b_ref, h_ref):
    acc = jax.lax.dot_general(
        x_ref[0], wt_ref[...], (((1,), (0,)), ((), ())),
        preferred_element_type=jnp.float32)
    h_ref[0] = acc + b_ref[...]


def _dft_topk_kernel(c_ref, s_ref, h_ref, cr_ref, ci_ref):
    h = h_ref[0]
    p = jax.lax.dot_general(c_ref[...], h, (((1,), (0,)), ((), ())),
                            preferred_element_type=jnp.float32,
                            precision=_PREC_HI)
    q = jax.lax.dot_general(s_ref[...], h, (((1,), (0,)), ((), ())),
                            preferred_element_type=jnp.float32,
                            precision=_PREC_HI)
    amp = p * p + q * q
    fio = jax.lax.broadcasted_iota(jnp.int32, amp.shape, 0)
    remaining = amp
    mask = jnp.zeros(amp.shape, dtype=jnp.bool_)
    for _ in range(_TOPK):
        m = jnp.max(remaining, axis=0, keepdims=True)
        ismax = remaining == m
        idx = jnp.min(jnp.where(ismax, fio, _FP), axis=0, keepdims=True)
        win = fio == idx
        mask = mask | win
        remaining = jnp.where(win, np.float32(-1.0), remaining)
    scale = jnp.where((fio == 0) | (fio == _N // 2),
                      np.float32(1.0 / _N), np.float32(2.0 / _N))
    coef = jnp.where(mask, scale, np.float32(0.0))
    cr_ref[0] = coef * p
    ci_ref[0] = coef * q


def _synth_kernel(ct_ref, st_ref, cr_ref, ci_ref, o_ref):
    oc = jax.lax.dot_general(ct_ref[...], cr_ref[0], (((1,), (0,)), ((), ())),
                             preferred_element_type=jnp.float32)
    os_ = jax.lax.dot_general(st_ref[...], ci_ref[0], (((1,), (0,)), ((), ())),
                              preferred_element_type=jnp.float32)
    o_ref[0] = oc + os_


def kernel(x, W, b):
    bsz, seq, in_dim = x.shape
    hid = W.shape[0]
    cosf, sinf = _basis()
    cost, sint = cosf.T, sinf.T
    wt = W.T
    b2 = b.reshape(1, hid)

    h = pl.pallas_call(
        _proj_kernel,
        grid=(bsz, seq // _ST),
        in_specs=[
            pl.BlockSpec((1, _ST, in_dim), lambda i, j: (i, j, 0)),
            pl.BlockSpec((in_dim, hid), lambda i, j: (0, 0)),
            pl.BlockSpec((1, hid), lambda i, j: (0, 0)),
        ],
        out_specs=pl.BlockSpec((1, _ST, hid), lambda i, j: (i, j, 0)),
        out_shape=jax.ShapeDtypeStruct((bsz, seq, hid), jnp.float32),
    )(x, wt, b2)

    fdt = jax.ShapeDtypeStruct((bsz, _FP, hid), jnp.float32)
    cr, ci = pl.pallas_call(
        _dft_topk_kernel,
        grid=(bsz, hid // _CT),
        in_specs=[
            pl.BlockSpec((_FP, seq), lambda i, j: (0, 0)),
            pl.BlockSpec((_FP, seq), lambda i, j: (0, 0)),
            pl.BlockSpec((1, seq, _CT), lambda i, j: (i, 0, j)),
        ],
        out_specs=[
            pl.BlockSpec((1, _FP, _CT), lambda i, j: (i, 0, j)),
            pl.BlockSpec((1, _FP, _CT), lambda i, j: (i, 0, j)),
        ],
        out_shape=[fdt, fdt],
    )(cosf, sinf, h)

    out = pl.pallas_call(
        _synth_kernel,
        grid=(bsz, seq // _ST),
        in_specs=[
            pl.BlockSpec((_ST, _FP), lambda i, j: (j, 0)),
            pl.BlockSpec((_ST, _FP), lambda i, j: (j, 0)),
            pl.BlockSpec((1, _FP, hid), lambda i, j: (i, 0, 0)),
            pl.BlockSpec((1, _FP, hid), lambda i, j: (i, 0, 0)),
        ],
        out_specs=pl.BlockSpec((1, _ST, hid), lambda i, j: (i, j, 0)),
        out_shape=jax.ShapeDtypeStruct((bsz, seq, hid), jnp.float32),
    )(cost, sint, cr, ci)
    return out


# speedup vs baseline: 13.8502x; 1.2712x over previous
import jax
import jax.numpy as jnp
import numpy as np
from jax.experimental import pallas as pl

_TOPK = 32
_N = 2048
_NF = _N // 2 + 1
_FP = 1040
_ST = 512
_CT = 256

_PREC_HI = jax.lax.Precision.HIGHEST
_PREC_MED = jax.lax.Precision.HIGHEST


def _basis():
    f = jnp.arange(_FP, dtype=jnp.int32)[:, None]
    s = jnp.arange(_N, dtype=jnp.int32)[None, :]
    m = (f * s) % _N
    ang = m.astype(jnp.float32) * np.float32(2.0 * np.pi / _N)
    valid = (f < _NF).astype(jnp.float32)
    return jnp.cos(ang) * valid, jnp.sin(ang) * valid


def _proj_kernel(x_ref, wt_ref, b_ref, h_ref):
    acc = jax.lax.dot_general(
        x_ref[0], wt_ref[...], (((1,), (0,)), ((), ())),
        preferred_element_type=jnp.float32)
    h_ref[0] = acc + b_ref[...]


def _dft_topk_kernel(c_ref, s_ref, h_ref, cr_ref, ci_ref):
    h = h_ref[0]
    p = jax.lax.dot_general(c_ref[...], h, (((1,), (0,)), ((), ())),
                            preferred_element_type=jnp.float32,
                            precision=_PREC_HI)
    q = jax.lax.dot_general(s_ref[...], h, (((1,), (0,)), ((), ())),
                            preferred_element_type=jnp.float32,
                            precision=_PREC_HI)
    amp = p * p + q * q
    fio = jax.lax.broadcasted_iota(jnp.int32, amp.shape, 0)
    a_int = jax.lax.bitcast_convert_type(amp, jnp.int32)
    lo = jnp.zeros((1, amp.shape[1]), jnp.int32)
    hi = jnp.max(a_int, axis=0, keepdims=True) + 1
    for _ in range(31):
        mid = lo + ((hi - lo) >> 1)
        cnt = jnp.sum((a_int >= mid).astype(jnp.int32), axis=0, keepdims=True)
        pred = cnt >= _TOPK
        lo = jnp.where(pred, mid, lo)
        hi = jnp.where(pred, hi, mid)
    t = lo
    gt = a_int > t
    eq = a_int == t
    r = eq.astype(jnp.int32)
    d = 1
    while d < amp.shape[0]:
        r = r + jnp.concatenate(
            [jnp.zeros((d, amp.shape[1]), jnp.int32), r[:-d]], axis=0)
        d *= 2
    need = _TOPK - jnp.sum(gt.astype(jnp.int32), axis=0, keepdims=True)
    mask = gt | (eq & (r <= need))
    scale = jnp.where((fio == 0) | (fio == _N // 2),
                      np.float32(1.0 / _N), np.float32(2.0 / _N))
    coef = jnp.where(mask, scale, np.float32(0.0))
    cr_ref[0] = coef * p
    ci_ref[0] = coef * q


def _synth_kernel(ct_ref, st_ref, cr_ref, ci_ref, o_ref):
    oc = jax.lax.dot_general(ct_ref[...], cr_ref[0], (((1,), (0,)), ((), ())),
                             preferred_element_type=jnp.float32)
    os_ = jax.lax.dot_general(st_ref[...], ci_ref[0], (((1,), (0,)), ((), ())),
                              preferred_element_type=jnp.float32)
    o_ref[0] = oc + os_


def kernel(x, W, b):
    bsz, seq, in_dim = x.shape
    hid = W.shape[0]
    cosf, sinf = _basis()
    cost, sint = cosf.T, sinf.T
    wt = W.T
    b2 = b.reshape(1, hid)

    h = pl.pallas_call(
        _proj_kernel,
        grid=(bsz, seq // _ST),
        in_specs=[
            pl.BlockSpec((1, _ST, in_dim), lambda i, j: (i, j, 0)),
            pl.BlockSpec((in_dim, hid), lambda i, j: (0, 0)),
            pl.BlockSpec((1, hid), lambda i, j: (0, 0)),
        ],
        out_specs=pl.BlockSpec((1, _ST, hid), lambda i, j: (i, j, 0)),
        out_shape=jax.ShapeDtypeStruct((bsz, seq, hid), jnp.float32),
    )(x, wt, b2)

    fdt = jax.ShapeDtypeStruct((bsz, _FP, hid), jnp.float32)
    cr, ci = pl.pallas_call(
        _dft_topk_kernel,
        grid=(bsz, hid // _CT),
        in_specs=[
            pl.BlockSpec((_FP, seq), lambda i, j: (0, 0)),
            pl.BlockSpec((_FP, seq), lambda i, j: (0, 0)),
            pl.BlockSpec((1, seq, _CT), lambda i, j: (i, 0, j)),
        ],
        out_specs=[
            pl.BlockSpec((1, _FP, _CT), lambda i, j: (i, 0, j)),
            pl.BlockSpec((1, _FP, _CT), lambda i, j: (i, 0, j)),
        ],
        out_shape=[fdt, fdt],
    )(cosf, sinf, h)

    out = pl.pallas_call(
        _synth_kernel,
        grid=(bsz, seq // _ST),
        in_specs=[
            pl.BlockSpec((_ST, _FP), lambda i, j: (j, 0)),
            pl.BlockSpec((_ST, _FP), lambda i, j: (j, 0)),
            pl.BlockSpec((1, _FP, hid), lambda i, j: (i, 0, 0)),
            pl.BlockSpec((1, _FP, hid), lambda i, j: (i, 0, 0)),
        ],
        out_specs=pl.BlockSpec((1, _ST, hid), lambda i, j: (i, j, 0)),
        out_shape=jax.ShapeDtypeStruct((bsz, seq, hid), jnp.float32),
    )(cost, sint, cr, ci)
    return out
